# baseline (speedup 1.0000x reference)
import jax
import jax.numpy as jnp
from jax import lax
from jax.experimental import pallas as pl
from jax.experimental.pallas import tpu as pltpu
from jax.experimental.pallas import tpu_sc as plsc

_N = 100000
_E = 3200000
_H = 16
_CK = 128
_NCHUNK = _E // _CK
_KD = 10
_GED = _KD * _CK
_NGD = _NCHUNK // _KD
_NW = 32
_QGD = _NGD // _NW
_RGD = _NGD - _QGD * _NW
_KE = 5
_GEE = _KE * _CK
_NGE = _NCHUNK // _KE
_QGE = _NGE // _NW
_RGE = _NGE - _QGE * _NW
_NP = 100352
_SL = _NP // 16
_ZF = _SL // _CK


def _sc_deg_body(ef, deg_out, idx, ones_v, zb, expv, deg_sh, isem, ssem):
    cid = lax.axis_index("c")
    sid = lax.axis_index("s")
    w = cid * 16 + sid

    def zb_store(i, carry):
        zb[pl.ds(i * 16, 16)] = jnp.zeros((16,), jnp.float32)
        return carry

    lax.fori_loop(0, _SL // 16, zb_store, 0)
    pltpu.sync_copy(zb, deg_sh.at[pl.ds(sid * _SL, _SL)])

    def ones_store(i, carry):
        ones_v[pl.ds(i * 16, 16)] = jnp.ones((16,), jnp.float32)
        return carry

    lax.fori_loop(0, _GED // 16, ones_store, 0)
    plsc.subcore_barrier()

    base = w * _QGD + jnp.minimum(w, _RGD)
    ng = _QGD + jnp.where(w < _RGD, 1, 0)

    def drain_s(buf):
        pltpu.make_async_copy(deg_out.at[0, pl.ds(0, _GED // _H)],
                              expv.at[pl.ds(0, _GED // _H)], ssem.at[buf]).wait()

    def load_idx(g, buf):
        pltpu.sync_copy(ef.at[pl.ds(_E + (base + g) * _GED, _GED)], idx.at[buf])

    def scatters(buf):
        pltpu.async_copy(ones_v, deg_sh.at[idx.at[buf]], ssem.at[buf], add=True)

    load_idx(0, 0)

    def body(g, carry):
        buf = lax.rem(g, 2)
        nbuf = 1 - buf

        @pl.when(g + 1 < ng)
        def _prefetch():
            @pl.when(g >= 1)
            def _():
                drain_s(nbuf)
            load_idx(g + 1, nbuf)

        scatters(buf)
        return carry

    lax.fori_loop(0, ng, body, 0)
    drain_s(0)
    drain_s(1)
    plsc.subcore_barrier()
    pltpu.sync_copy(deg_sh.at[pl.ds(sid * _SL, _SL)], zb)

    def expand(i, carry):
        v = zb[pl.ds(i * 16, 16)]
        for l in range(16):
            expv[i * 16 + l, :] = jnp.full((16,), v[l], jnp.float32)
        return carry

    lax.fori_loop(0, _SL // 16, expand, 0)
    pltpu.sync_copy(expv, deg_out.at[cid, pl.ds(sid * _SL, _SL)])


_sc_deg = pl.kernel(
    _sc_deg_body,
    out_type=jax.ShapeDtypeStruct((2, _NP, _H), jnp.float32),
    mesh=plsc.VectorSubcoreMesh(core_axis_name="c", subcore_axis_name="s"),
    compiler_params=pltpu.CompilerParams(use_tc_tiling_on_sc=False),
    scratch_types=[
        pltpu.VMEM((2, _GED), jnp.int32),
        pltpu.VMEM((_GED,), jnp.float32),
        pltpu.VMEM((_SL,), jnp.float32),
        pltpu.VMEM((_SL, _H), jnp.float32),
        pltpu.VMEM_SHARED((_NP,), jnp.float32),
        pltpu.SemaphoreType.DMA((2,)),
        pltpu.SemaphoreType.DMA((2,)),
    ],
)


def _sc_edge_body(ef, hn, acc_out, idx, rows, acc_sh, gsem, ssem):
    cid = lax.axis_index("c")
    sid = lax.axis_index("s")
    w = cid * 16 + sid

    def zrow(i, carry):
        rows[0, i, :] = jnp.zeros((16,), jnp.float32)
        return carry

    lax.fori_loop(0, _CK, zrow, 0)
    r0 = sid * _SL

    def zacc(k, carry):
        pltpu.sync_copy(rows.at[0, pl.ds(0, _CK)], acc_sh.at[pl.ds(r0 + k * _CK, _CK)])
        return carry

    lax.fori_loop(0, _ZF, zacc, 0)
    plsc.subcore_barrier()

    base = w * _QGE + jnp.minimum(w, _RGE)
    ng = _QGE + jnp.where(w < _RGE, 1, 0)

    def drain(sem, buf):
        pltpu.make_async_copy(acc_out.at[0, pl.ds(0, _GEE)],
                              rows.at[buf], sem.at[buf]).wait()

    def load_idx(g, buf):
        pltpu.sync_copy(ef.at[pl.ds((base + g) * _GEE, _GEE)], idx.at[buf, 0])
        pltpu.sync_copy(ef.at[pl.ds(_E + (base + g) * _GEE, _GEE)], idx.at[buf, 1])

    def gathers(buf):
        pltpu.async_copy(hn.at[idx.at[buf, 0]], rows.at[buf], gsem.at[buf])

    def scatters(buf):
        pltpu.async_copy(rows.at[buf], acc_sh.at[idx.at[buf, 1]],
                         ssem.at[buf], add=True)

    load_idx(0, 0)
    gathers(0)

    def body(g, carry):
        buf = lax.rem(g, 2)
        nbuf = 1 - buf

        @pl.when(g + 1 < ng)
        def _prefetch():
            @pl.when(g >= 1)
            def _():
                drain(ssem, nbuf)
            load_idx(g + 1, nbuf)
            gathers(nbuf)

        drain(gsem, buf)
        scatters(buf)
        return carry

    lax.fori_loop(0, ng, body, 0)
    drain(ssem, 0)
    drain(ssem, 1)
    plsc.subcore_barrier()
    pltpu.sync_copy(acc_sh.at[pl.ds(r0, _SL)],
                    acc_out.at[cid, pl.ds(r0, _SL)])


_sc_edge = pl.kernel(
    _sc_edge_body,
    out_type=jax.ShapeDtypeStruct((2, _NP, _H), jnp.float32),
    mesh=plsc.VectorSubcoreMesh(core_axis_name="c", subcore_axis_name="s"),
    compiler_params=pltpu.CompilerParams(use_tc_tiling_on_sc=False),
    scratch_types=[
        pltpu.VMEM((2, 2, _GEE), jnp.int32),
        pltpu.VMEM((2, _GEE, _H), jnp.float32),
        pltpu.VMEM_SHARED((_NP, _H), jnp.float32),
        pltpu.SemaphoreType.DMA((2,)),
        pltpu.SemaphoreType.DMA((2,)),
    ],
)

_RB = 2048
_GRID = _NP // _RB

_SQRT_HALF = 0.7071067811865476


def _gelu(t):
    return 0.5 * t * (1.0 + lax.erf(t * _SQRT_HALF))


def _tc_layer0_body(x_ref, w0_ref, b0_ref, g0_ref, bb0_ref, wg_ref, h1_ref, hp_ref):
    h = jnp.dot(x_ref[...], w0_ref[...], preferred_element_type=jnp.float32)
    h = h + b0_ref[...]
    m = jnp.mean(h, axis=1, keepdims=True)
    v = jnp.mean((h - m) ** 2, axis=1, keepdims=True)
    h = (h - m) / jnp.sqrt(v + 1e-5) * g0_ref[...] + bb0_ref[...]
    h1 = _gelu(h)
    h1_ref[...] = h1
    hp_ref[...] = jnp.dot(h1, wg_ref[...], preferred_element_type=jnp.float32)


_tc_layer0 = pl.pallas_call(
    _tc_layer0_body,
    grid=(_GRID,),
    in_specs=[
        pl.BlockSpec((_RB, 128), lambda i: (i, 0)),
        pl.BlockSpec((128, _H), lambda i: (0, 0)),
        pl.BlockSpec((1, _H), lambda i: (0, 0)),
        pl.BlockSpec((1, _H), lambda i: (0, 0)),
        pl.BlockSpec((1, _H), lambda i: (0, 0)),
        pl.BlockSpec((_H, _H), lambda i: (0, 0)),
    ],
    out_specs=[
        pl.BlockSpec((_RB, _H), lambda i: (i, 0)),
        pl.BlockSpec((_RB, _H), lambda i: (i, 0)),
    ],
    out_shape=[
        jax.ShapeDtypeStruct((_N, _H), jnp.float32),
        jax.ShapeDtypeStruct((_N, _H), jnp.float32),
    ],
)


def _tc_hn_body(d_ref, hp_ref, hn_ref):
    dis = lax.rsqrt(d_ref[0] + d_ref[1] + 1.0)
    hn_ref[...] = hp_ref[...] * dis


_tc_hn = pl.pallas_call(
    _tc_hn_body,
    grid=(_GRID,),
    in_specs=[
        pl.BlockSpec((2, _RB, _H), lambda i: (0, i, 0)),
        pl.BlockSpec((_RB, _H), lambda i: (i, 0)),
    ],
    out_specs=pl.BlockSpec((_RB, _H), lambda i: (i, 0)),
    out_shape=jax.ShapeDtypeStruct((_N, _H), jnp.float32),
)


def _tc_final_body(acc_ref, d_ref, hn_ref, h1_ref, bg_ref, g1_ref,
                   b1_ref, w2_ref, b2_ref, out_ref):
    a = acc_ref[0] + acc_ref[1]
    dis = lax.rsqrt(d_ref[0] + d_ref[1] + 1.0)
    z = (a + hn_ref[...]) * dis + bg_ref[...]
    m = jnp.mean(z, axis=1, keepdims=True)
    v = jnp.mean((z - m) ** 2, axis=1, keepdims=True)
    t = (z - m) / jnp.sqrt(v + 1e-5) * g1_ref[...] + b1_ref[...]
    t = _gelu(t)
    h = t + h1_ref[...]
    out_ref[...] = jnp.dot(h, w2_ref[...], preferred_element_type=jnp.float32) + b2_ref[...]


_tc_final = pl.pallas_call(
    _tc_final_body,
    grid=(_GRID,),
    in_specs=[
        pl.BlockSpec((2, _RB, _H), lambda i: (0, i, 0)),
        pl.BlockSpec((2, _RB, _H), lambda i: (0, i, 0)),
        pl.BlockSpec((_RB, _H), lambda i: (i, 0)),
        pl.BlockSpec((_RB, _H), lambda i: (i, 0)),
        pl.BlockSpec((1, _H), lambda i: (0, 0)),
        pl.BlockSpec((1, _H), lambda i: (0, 0)),
        pl.BlockSpec((1, _H), lambda i: (0, 0)),
        pl.BlockSpec((_H, 128), lambda i: (0, 0)),
        pl.BlockSpec((1, 128), lambda i: (0, 0)),
    ],
    out_specs=pl.BlockSpec((_RB, 128), lambda i: (i, 0)),
    out_shape=jax.ShapeDtypeStruct((_N, 128), jnp.float32),
)


def kernel(x, edge_index, W0, b0, ln0_g, ln0_b, Wg, bg, ln1_g, ln1_b, W2, b2):
    ef = edge_index.reshape(2 * _E)
    degw = _sc_deg(ef)
    h1, hp = _tc_layer0(x, W0, b0.reshape(1, _H), ln0_g.reshape(1, _H),
                        ln0_b.reshape(1, _H), Wg)
    hn = _tc_hn(degw, hp)
    acc = _sc_edge(ef, hn)
    out = _tc_final(acc, degw, hn, h1, bg.reshape(1, _H),
                    ln1_g.reshape(1, _H), ln1_b.reshape(1, _H),
                    W2, b2.reshape(1, 128))
    return out


# speedup vs baseline: 1.1354x; 1.1354x over previous
import jax
import jax.numpy as jnp
from jax import lax
from jax.experimental import pallas as pl
from jax.experimental.pallas import tpu as pltpu
from jax.experimental.pallas import tpu_sc as plsc

_N = 100000
_E = 3200000
_H = 16
_CK = 128
_NCHUNK = _E // _CK
_KD = 10
_GED = _KD * _CK
_NGD = _NCHUNK // _KD
_NW = 32
_QGD = _NGD // _NW
_RGD = _NGD - _QGD * _NW
_KE = 5
_GEE = _KE * _CK
_NGE = _NCHUNK // _KE
_QGE = _NGE // _NW
_RGE = _NGE - _QGE * _NW
_NP = 100352
_SL = _NP // 16
_ZF = _SL // _CK


def _sc_deg_body(ef, deg_out, idx, ones_v, zb, deg_sh, isem, ssem):
    cid = lax.axis_index("c")
    sid = lax.axis_index("s")
    w = cid * 16 + sid

    def zb_store(i, carry):
        zb[pl.ds(i * 16, 16)] = jnp.zeros((16,), jnp.float32)
        return carry

    lax.fori_loop(0, _SL // 16, zb_store, 0)
    pltpu.sync_copy(zb, deg_sh.at[pl.ds(sid * _SL, _SL)])

    def ones_store(i, carry):
        ones_v[pl.ds(i * 16, 16)] = jnp.ones((16,), jnp.float32)
        return carry

    lax.fori_loop(0, _GED // 16, ones_store, 0)
    plsc.subcore_barrier()

    base = w * _QGD + jnp.minimum(w, _RGD)
    ng = _QGD + jnp.where(w < _RGD, 1, 0)

    def drain_s(buf):
        pltpu.make_async_copy(deg_out.at[pl.ds(0, _GED)],
                              zb.at[pl.ds(0, _GED)], ssem.at[buf]).wait()

    def load_idx(g, buf):
        pltpu.sync_copy(ef.at[pl.ds(_E + (base + g) * _GED, _GED)], idx.at[buf])

    def scatters(buf):
        pltpu.async_copy(ones_v, deg_sh.at[idx.at[buf]], ssem.at[buf], add=True)

    load_idx(0, 0)

    def body(g, carry):
        buf = lax.rem(g, 2)
        nbuf = 1 - buf

        @pl.when(g + 1 < ng)
        def _prefetch():
            @pl.when(g >= 1)
            def _():
                drain_s(nbuf)
            load_idx(g + 1, nbuf)

        scatters(buf)
        return carry

    lax.fori_loop(0, ng, body, 0)
    drain_s(0)
    drain_s(1)
    plsc.subcore_barrier()
    pltpu.sync_copy(deg_sh.at[pl.ds(sid * _SL, _SL)],
                    deg_out.at[pl.ds(cid * _NP + sid * _SL, _SL)])


_sc_deg = pl.kernel(
    _sc_deg_body,
    out_type=jax.ShapeDtypeStruct((2 * _NP,), jnp.float32),
    mesh=plsc.VectorSubcoreMesh(core_axis_name="c", subcore_axis_name="s"),
    compiler_params=pltpu.CompilerParams(use_tc_tiling_on_sc=False),
    scratch_types=[
        pltpu.VMEM((2, _GED), jnp.int32),
        pltpu.VMEM((_GED,), jnp.float32),
        pltpu.VMEM((_SL,), jnp.float32),
        pltpu.VMEM_SHARED((_NP,), jnp.float32),
        pltpu.SemaphoreType.DMA((2,)),
        pltpu.SemaphoreType.DMA((2,)),
    ],
)


def _sc_edge_body(ef, hn, acc_out, idx, rows, acc_sh, gsem, ssem):
    cid = lax.axis_index("c")
    sid = lax.axis_index("s")
    w = cid * 16 + sid

    def zrow(i, carry):
        rows[0, i, :] = jnp.zeros((16,), jnp.float32)
        return carry

    lax.fori_loop(0, _CK, zrow, 0)
    r0 = sid * _SL

    def zacc(k, carry):
        pltpu.sync_copy(rows.at[0, pl.ds(0, _CK)], acc_sh.at[pl.ds(r0 + k * _CK, _CK)])
        return carry

    lax.fori_loop(0, _ZF, zacc, 0)
    plsc.subcore_barrier()

    base = w * _QGE + jnp.minimum(w, _RGE)
    ng = _QGE + jnp.where(w < _RGE, 1, 0)

    def drain(sem, buf):
        pltpu.make_async_copy(acc_out.at[0, pl.ds(0, _GEE)],
                              rows.at[buf], sem.at[buf]).wait()

    def load_idx(g, buf):
        pltpu.sync_copy(ef.at[pl.ds((base + g) * _GEE, _GEE)], idx.at[buf, 0])
        pltpu.sync_copy(ef.at[pl.ds(_E + (base + g) * _GEE, _GEE)], idx.at[buf, 1])

    def gathers(buf):
        pltpu.async_copy(hn.at[idx.at[buf, 0]], rows.at[buf], gsem.at[buf])

    def scatters(buf):
        pltpu.async_copy(rows.at[buf], acc_sh.at[idx.at[buf, 1]],
                         ssem.at[buf], add=True)

    load_idx(0, 0)
    gathers(0)

    def body(g, carry):
        buf = lax.rem(g, 2)
        nbuf = 1 - buf

        @pl.when(g + 1 < ng)
        def _prefetch():
            @pl.when(g >= 1)
            def _():
                drain(ssem, nbuf)
            load_idx(g + 1, nbuf)
            gathers(nbuf)

        drain(gsem, buf)
        scatters(buf)
        return carry

    lax.fori_loop(0, ng, body, 0)
    drain(ssem, 0)
    drain(ssem, 1)
    plsc.subcore_barrier()
    pltpu.sync_copy(acc_sh.at[pl.ds(r0, _SL)],
                    acc_out.at[cid, pl.ds(r0, _SL)])


_sc_edge = pl.kernel(
    _sc_edge_body,
    out_type=jax.ShapeDtypeStruct((2, _NP, _H), jnp.float32),
    mesh=plsc.VectorSubcoreMesh(core_axis_name="c", subcore_axis_name="s"),
    compiler_params=pltpu.CompilerParams(use_tc_tiling_on_sc=False),
    scratch_types=[
        pltpu.VMEM((2, 2, _GEE), jnp.int32),
        pltpu.VMEM((2, _GEE, _H), jnp.float32),
        pltpu.VMEM_SHARED((_NP, _H), jnp.float32),
        pltpu.SemaphoreType.DMA((2,)),
        pltpu.SemaphoreType.DMA((2,)),
    ],
)

_RB = 2048
_GRID = _NP // _RB

_SQRT_HALF = 0.7071067811865476


def _gelu(t):
    return 0.5 * t * (1.0 + lax.erf(t * _SQRT_HALF))


def _tc_layer0_body(x_ref, w0_ref, b0_ref, g0_ref, bb0_ref, wg_ref, h1_ref, hp_ref):
    h = jnp.dot(x_ref[...], w0_ref[...], preferred_element_type=jnp.float32)
    h = h + b0_ref[...]
    m = jnp.mean(h, axis=1, keepdims=True)
    v = jnp.mean((h - m) ** 2, axis=1, keepdims=True)
    h = (h - m) / jnp.sqrt(v + 1e-5) * g0_ref[...] + bb0_ref[...]
    h1 = _gelu(h)
    h1_ref[...] = h1
    hp_ref[...] = jnp.dot(h1, wg_ref[...], preferred_element_type=jnp.float32)


_tc_layer0 = pl.pallas_call(
    _tc_layer0_body,
    grid=(_GRID,),
    in_specs=[
        pl.BlockSpec((_RB, 128), lambda i: (i, 0)),
        pl.BlockSpec((128, _H), lambda i: (0, 0)),
        pl.BlockSpec((1, _H), lambda i: (0, 0)),
        pl.BlockSpec((1, _H), lambda i: (0, 0)),
        pl.BlockSpec((1, _H), lambda i: (0, 0)),
        pl.BlockSpec((_H, _H), lambda i: (0, 0)),
    ],
    out_specs=[
        pl.BlockSpec((_RB, _H), lambda i: (i, 0)),
        pl.BlockSpec((_RB, _H), lambda i: (i, 0)),
    ],
    out_shape=[
        jax.ShapeDtypeStruct((_N, _H), jnp.float32),
        jax.ShapeDtypeStruct((_N, _H), jnp.float32),
    ],
)


def _dis_col(d):
    t = d[0, :] + d[1, :] + 1.0
    return lax.rsqrt(t).reshape(_RB, 1)


def _tc_hn_body(d_ref, hp_ref, hn_ref):
    hn_ref[...] = hp_ref[...] * _dis_col(d_ref[...])


_tc_hn = pl.pallas_call(
    _tc_hn_body,
    grid=(_GRID,),
    in_specs=[
        pl.BlockSpec((2, _RB), lambda i: (0, i)),
        pl.BlockSpec((_RB, _H), lambda i: (i, 0)),
    ],
    out_specs=pl.BlockSpec((_RB, _H), lambda i: (i, 0)),
    out_shape=jax.ShapeDtypeStruct((_N, _H), jnp.float32),
)


def _tc_final_body(acc_ref, d_ref, hn_ref, h1_ref, bg_ref, g1_ref,
                   b1_ref, w2_ref, b2_ref, out_ref):
    a = acc_ref[0] + acc_ref[1]
    dis = _dis_col(d_ref[...])
    z = (a + hn_ref[...]) * dis + bg_ref[...]
    m = jnp.mean(z, axis=1, keepdims=True)
    v = jnp.mean((z - m) ** 2, axis=1, keepdims=True)
    t = (z - m) / jnp.sqrt(v + 1e-5) * g1_ref[...] + b1_ref[...]
    t = _gelu(t)
    h = t + h1_ref[...]
    out_ref[...] = jnp.dot(h, w2_ref[...], preferred_element_type=jnp.float32) + b2_ref[...]


_tc_final = pl.pallas_call(
    _tc_final_body,
    grid=(_GRID,),
    in_specs=[
        pl.BlockSpec((2, _RB, _H), lambda i: (0, i, 0)),
        pl.BlockSpec((2, _RB), lambda i: (0, i)),
        pl.BlockSpec((_RB, _H), lambda i: (i, 0)),
        pl.BlockSpec((_RB, _H), lambda i: (i, 0)),
        pl.BlockSpec((1, _H), lambda i: (0, 0)),
        pl.BlockSpec((1, _H), lambda i: (0, 0)),
        pl.BlockSpec((1, _H), lambda i: (0, 0)),
        pl.BlockSpec((_H, 128), lambda i: (0, 0)),
        pl.BlockSpec((1, 128), lambda i: (0, 0)),
    ],
    out_specs=pl.BlockSpec((_RB, 128), lambda i: (i, 0)),
    out_shape=jax.ShapeDtypeStruct((_N, 128), jnp.float32),
)


def kernel(x, edge_index, W0, b0, ln0_g, ln0_b, Wg, bg, ln1_g, ln1_b, W2, b2):
    ef = edge_index.reshape(2 * _E)
    deg = _sc_deg(ef)
    h1, hp = _tc_layer0(x, W0, b0.reshape(1, _H), ln0_g.reshape(1, _H),
                        ln0_b.reshape(1, _H), Wg)
    deg2 = deg.reshape(2, _NP)
    hn = _tc_hn(deg2, hp)
    acc = _sc_edge(ef, hn)
    out = _tc_final(acc, deg2, hn, h1, bg.reshape(1, _H),
                    ln1_g.reshape(1, _H), ln1_b.reshape(1, _H),
                    W2, b2.reshape(1, 128))
    return out


# speedup vs baseline: 1.3949x; 1.2285x over previous
import jax
import jax.numpy as jnp
from jax import lax
from jax.experimental import pallas as pl
from jax.experimental.pallas import tpu as pltpu
from jax.experimental.pallas import tpu_sc as plsc

_N = 100000
_E = 3200000
_H = 16
_CK = 128
_NCHUNK = _E // _CK
_KD = 10
_GED = _KD * _CK
_NGD = _NCHUNK // _KD
_NW = 32
_QGD = _NGD // _NW
_RGD = _NGD - _QGD * _NW
_KE = 5
_GEE = _KE * _CK
_NGE = _NCHUNK // _KE
_QGE = _NGE // _NW
_RGE = _NGE - _QGE * _NW
_NP = 100352
_SL = _NP // 16
_ZF = _SL // _CK


def _sc_deg_body(ef, deg_out, idx, ones_v, zb, deg_sh, isem, ssem):
    cid = lax.axis_index("c")
    sid = lax.axis_index("s")
    w = cid * 16 + sid

    def zb_store(i, carry):
        zb[pl.ds(i * 16, 16)] = jnp.zeros((16,), jnp.float32)
        return carry

    lax.fori_loop(0, _SL // 16, zb_store, 0)
    pltpu.sync_copy(zb, deg_sh.at[pl.ds(sid * _SL, _SL)])

    def ones_store(i, carry):
        ones_v[pl.ds(i * 16, 16)] = jnp.ones((16,), jnp.float32)
        return carry

    lax.fori_loop(0, _GED // 16, ones_store, 0)
    plsc.subcore_barrier()

    base = w * _QGD + jnp.minimum(w, _RGD)
    ng = _QGD + jnp.where(w < _RGD, 1, 0)

    def drain_s(p):
        pltpu.make_async_copy(deg_out.at[pl.ds(0, _GED)],
                              zb.at[pl.ds(0, _GED)], ssem.at[p]).wait()

    def drain_i(buf):
        pltpu.make_async_copy(ef.at[pl.ds(0, _GED)], idx.at[buf],
                              isem.at[buf]).wait()

    def load_idx(g, buf):
        pltpu.async_copy(ef.at[pl.ds(_E + (base + g) * _GED, _GED)],
                         idx.at[buf], isem.at[buf])

    def scatters(g, buf):
        pltpu.async_copy(ones_v, deg_sh.at[idx.at[buf]],
                         ssem.at[lax.rem(g, 2)], add=True)

    load_idx(0, 0)
    load_idx(1, 1)

    def body(g, carry):
        b0 = lax.rem(g, 3)
        b2 = lax.rem(g + 2, 3)

        @pl.when(g + 2 < ng)
        def _prefetch():
            @pl.when(g >= 1)
            def _():
                drain_s(lax.rem(g - 1, 2))
            load_idx(g + 2, b2)

        drain_i(b0)
        scatters(g, b0)
        return carry

    lax.fori_loop(0, ng, body, 0)
    p = lax.rem(ng - 1, 2)
    drain_s(p)
    drain_s(p)
    drain_s(1 - p)
    plsc.subcore_barrier()
    pltpu.sync_copy(deg_sh.at[pl.ds(sid * _SL, _SL)],
                    deg_out.at[pl.ds(cid * _NP + sid * _SL, _SL)])


_sc_deg = pl.kernel(
    _sc_deg_body,
    out_type=jax.ShapeDtypeStruct((2 * _NP,), jnp.float32),
    mesh=plsc.VectorSubcoreMesh(core_axis_name="c", subcore_axis_name="s"),
    compiler_params=pltpu.CompilerParams(use_tc_tiling_on_sc=False),
    scratch_types=[
        pltpu.VMEM((3, _GED), jnp.int32),
        pltpu.VMEM((_GED,), jnp.float32),
        pltpu.VMEM((_SL,), jnp.float32),
        pltpu.VMEM_SHARED((_NP,), jnp.float32),
        pltpu.SemaphoreType.DMA((3,)),
        pltpu.SemaphoreType.DMA((2,)),
    ],
)


def _sc_edge_body(ef, hn, acc_out, idx, rows, acc_sh, gsem, ssem, isem):
    cid = lax.axis_index("c")
    sid = lax.axis_index("s")
    w = cid * 16 + sid

    def zrow(i, carry):
        rows[0, i, :] = jnp.zeros((16,), jnp.float32)
        return carry

    lax.fori_loop(0, _CK, zrow, 0)
    r0 = sid * _SL

    def zacc(k, carry):
        pltpu.sync_copy(rows.at[0, pl.ds(0, _CK)], acc_sh.at[pl.ds(r0 + k * _CK, _CK)])
        return carry

    lax.fori_loop(0, _ZF, zacc, 0)
    plsc.subcore_barrier()

    base = w * _QGE + jnp.minimum(w, _RGE)
    ng = _QGE + jnp.where(w < _RGE, 1, 0)

    def drain(sem, p):
        pltpu.make_async_copy(acc_out.at[0, pl.ds(0, _GEE)],
                              rows.at[p], sem.at[p]).wait()

    def drain_i(buf):
        for h in range(2):
            pltpu.make_async_copy(ef.at[pl.ds(0, _GEE)], idx.at[buf, h],
                                  isem.at[buf]).wait()

    def load_idx(g, buf):
        pltpu.async_copy(ef.at[pl.ds((base + g) * _GEE, _GEE)],
                         idx.at[buf, 0], isem.at[buf])
        pltpu.async_copy(ef.at[pl.ds(_E + (base + g) * _GEE, _GEE)],
                         idx.at[buf, 1], isem.at[buf])

    def gathers(ib, rb):
        pltpu.async_copy(hn.at[idx.at[ib, 0]], rows.at[rb], gsem.at[rb])

    def scatters(ib, rb):
        pltpu.async_copy(rows.at[rb], acc_sh.at[idx.at[ib, 1]],
                         ssem.at[rb], add=True)

    load_idx(0, 0)
    load_idx(1, 1)
    drain_i(0)
    gathers(0, 0)

    def body(g, carry):
        b0 = lax.rem(g, 3)
        b1 = lax.rem(g + 1, 3)
        b2 = lax.rem(g + 2, 3)
        rb = lax.rem(g, 2)
        rn = 1 - rb

        @pl.when(g + 1 < ng)
        def _prefetch():
            @pl.when(g >= 1)
            def _():
                drain(ssem, rn)

            @pl.when(g + 2 < ng)
            def _():
                load_idx(g + 2, b2)
            drain_i(b1)
            gathers(b1, rn)

        drain(gsem, rb)
        scatters(b0, rb)
        return carry

    lax.fori_loop(0, ng, body, 0)
    drain(ssem, 0)
    drain(ssem, 1)
    plsc.subcore_barrier()
    pltpu.sync_copy(acc_sh.at[pl.ds(r0, _SL)],
                    acc_out.at[cid, pl.ds(r0, _SL)])


_sc_edge = pl.kernel(
    _sc_edge_body,
    out_type=jax.ShapeDtypeStruct((2, _NP, _H), jnp.float32),
    mesh=plsc.VectorSubcoreMesh(core_axis_name="c", subcore_axis_name="s"),
    compiler_params=pltpu.CompilerParams(use_tc_tiling_on_sc=False),
    scratch_types=[
        pltpu.VMEM((3, 2, _GEE), jnp.int32),
        pltpu.VMEM((2, _GEE, _H), jnp.float32),
        pltpu.VMEM_SHARED((_NP, _H), jnp.float32),
        pltpu.SemaphoreType.DMA((2,)),
        pltpu.SemaphoreType.DMA((2,)),
        pltpu.SemaphoreType.DMA((3,)),
    ],
)

_RB = 2048
_GRID = _NP // _RB

_SQRT_HALF = 0.7071067811865476


def _gelu(t):
    return 0.5 * t * (1.0 + lax.erf(t * _SQRT_HALF))


def _tc_layer0_body(x_ref, w0_ref, b0_ref, g0_ref, bb0_ref, wg_ref, h1_ref, hp_ref):
    h = jnp.dot(x_ref[...], w0_ref[...], preferred_element_type=jnp.float32)
    h = h + b0_ref[...]
    m = jnp.mean(h, axis=1, keepdims=True)
    v = jnp.mean((h - m) ** 2, axis=1, keepdims=True)
    h = (h - m) / jnp.sqrt(v + 1e-5) * g0_ref[...] + bb0_ref[...]
    h1 = _gelu(h)
    h1_ref[...] = h1
    hp_ref[...] = jnp.dot(h1, wg_ref[...], preferred_element_type=jnp.float32)


_tc_layer0 = pl.pallas_call(
    _tc_layer0_body,
    grid=(_GRID,),
    in_specs=[
        pl.BlockSpec((_RB, 128), lambda i: (i, 0)),
        pl.BlockSpec((128, _H), lambda i: (0, 0)),
        pl.BlockSpec((1, _H), lambda i: (0, 0)),
        pl.BlockSpec((1, _H), lambda i: (0, 0)),
        pl.BlockSpec((1, _H), lambda i: (0, 0)),
        pl.BlockSpec((_H, _H), lambda i: (0, 0)),
    ],
    out_specs=[
        pl.BlockSpec((_RB, _H), lambda i: (i, 0)),
        pl.BlockSpec((_RB, _H), lambda i: (i, 0)),
    ],
    out_shape=[
        jax.ShapeDtypeStruct((_N, _H), jnp.float32),
        jax.ShapeDtypeStruct((_N, _H), jnp.float32),
    ],
)


def _dis_col(d):
    t = d[0, :] + d[1, :] + 1.0
    return lax.rsqrt(t).reshape(_RB, 1)


def _tc_hn_body(d_ref, hp_ref, hn_ref):
    hn_ref[...] = hp_ref[...] * _dis_col(d_ref[...])


_tc_hn = pl.pallas_call(
    _tc_hn_body,
    grid=(_GRID,),
    in_specs=[
        pl.BlockSpec((2, _RB), lambda i: (0, i)),
        pl.BlockSpec((_RB, _H), lambda i: (i, 0)),
    ],
    out_specs=pl.BlockSpec((_RB, _H), lambda i: (i, 0)),
    out_shape=jax.ShapeDtypeStruct((_N, _H), jnp.float32),
)


def _tc_final_body(acc_ref, d_ref, hn_ref, h1_ref, bg_ref, g1_ref,
                   b1_ref, w2_ref, b2_ref, out_ref):
    a = acc_ref[0] + acc_ref[1]
    dis = _dis_col(d_ref[...])
    z = (a + hn_ref[...]) * dis + bg_ref[...]
    m = jnp.mean(z, axis=1, keepdims=True)
    v = jnp.mean((z - m) ** 2, axis=1, keepdims=True)
    t = (z - m) / jnp.sqrt(v + 1e-5) * g1_ref[...] + b1_ref[...]
    t = _gelu(t)
    h = t + h1_ref[...]
    out_ref[...] = jnp.dot(h, w2_ref[...], preferred_element_type=jnp.float32) + b2_ref[...]


_tc_final = pl.pallas_call(
    _tc_final_body,
    grid=(_GRID,),
    in_specs=[
        pl.BlockSpec((2, _RB, _H), lambda i: (0, i, 0)),
        pl.BlockSpec((2, _RB), lambda i: (0, i)),
        pl.BlockSpec((_RB, _H), lambda i: (i, 0)),
        pl.BlockSpec((_RB, _H), lambda i: (i, 0)),
        pl.BlockSpec((1, _H), lambda i: (0, 0)),
        pl.BlockSpec((1, _H), lambda i: (0, 0)),
        pl.BlockSpec((1, _H), lambda i: (0, 0)),
        pl.BlockSpec((_H, 128), lambda i: (0, 0)),
        pl.BlockSpec((1, 128), lambda i: (0, 0)),
    ],
    out_specs=pl.BlockSpec((_RB, 128), lambda i: (i, 0)),
    out_shape=jax.ShapeDtypeStruct((_N, 128), jnp.float32),
)


def kernel(x, edge_index, W0, b0, ln0_g, ln0_b, Wg, bg, ln1_g, ln1_b, W2, b2):
    ef = edge_index.reshape(2 * _E)
    deg = _sc_deg(ef)
    h1, hp = _tc_layer0(x, W0, b0.reshape(1, _H), ln0_g.reshape(1, _H),
                        ln0_b.reshape(1, _H), Wg)
    deg2 = deg.reshape(2, _NP)
    hn = _tc_hn(deg2, hp)
    acc = _sc_edge(ef, hn)
    out = _tc_final(acc, deg2, hn, h1, bg.reshape(1, _H),
                    ln1_g.reshape(1, _H), ln1_b.reshape(1, _H),
                    W2, b2.reshape(1, 128))
    return out


# speedup vs baseline: 1.4670x; 1.0517x over previous
import jax
import jax.numpy as jnp
from jax import lax
from jax.experimental import pallas as pl
from jax.experimental.pallas import tpu as pltpu
from jax.experimental.pallas import tpu_sc as plsc

_N = 100000
_E = 3200000
_H = 16
_CK = 128
_NCHUNK = _E // _CK
_KD = 10
_GED = _KD * _CK
_NGD = _NCHUNK // _KD
_NW = 32
_QGD = _NGD // _NW
_RGD = _NGD - _QGD * _NW
_KE = 4
_GEE = _KE * _CK
_NGE = _NCHUNK // _KE
_QGE = _NGE // _NW
_RGE = _NGE - _QGE * _NW
_NP = 100352
_SL = _NP // 16
_ZF = _SL // _CK


def _sc_deg_body(ef, deg_out, idx, ones_v, zb, deg_sh, isem, ssem):
    cid = lax.axis_index("c")
    sid = lax.axis_index("s")
    w = cid * 16 + sid

    def zb_store(i, carry):
        zb[pl.ds(i * 16, 16)] = jnp.zeros((16,), jnp.float32)
        return carry

    lax.fori_loop(0, _SL // 16, zb_store, 0)
    pltpu.sync_copy(zb, deg_sh.at[pl.ds(sid * _SL, _SL)])

    def ones_store(i, carry):
        ones_v[pl.ds(i * 16, 16)] = jnp.ones((16,), jnp.float32)
        return carry

    lax.fori_loop(0, _GED // 16, ones_store, 0)
    plsc.subcore_barrier()

    base = w * _QGD + jnp.minimum(w, _RGD)
    ng = _QGD + jnp.where(w < _RGD, 1, 0)

    def drain_s(p):
        pltpu.make_async_copy(deg_out.at[pl.ds(0, _GED)],
                              zb.at[pl.ds(0, _GED)], ssem.at[p]).wait()

    def drain_i(buf):
        pltpu.make_async_copy(ef.at[pl.ds(0, _GED)], idx.at[buf],
                              isem.at[buf]).wait()

    def load_idx(g, buf):
        pltpu.async_copy(ef.at[pl.ds(_E + (base + g) * _GED, _GED)],
                         idx.at[buf], isem.at[buf])

    def scatters(g, buf):
        pltpu.async_copy(ones_v, deg_sh.at[idx.at[buf]],
                         ssem.at[lax.rem(g, 2)], add=True)

    load_idx(0, 0)
    load_idx(1, 1)

    def body(g, carry):
        b0 = lax.rem(g, 3)
        b2 = lax.rem(g + 2, 3)

        @pl.when(g + 2 < ng)
        def _prefetch():
            @pl.when(g >= 1)
            def _():
                drain_s(lax.rem(g - 1, 2))
            load_idx(g + 2, b2)

        drain_i(b0)
        scatters(g, b0)
        return carry

    lax.fori_loop(0, ng, body, 0)
    p = lax.rem(ng - 1, 2)
    drain_s(p)
    drain_s(p)
    drain_s(1 - p)
    plsc.subcore_barrier()
    pltpu.sync_copy(deg_sh.at[pl.ds(sid * _SL, _SL)],
                    deg_out.at[pl.ds(cid * _NP + sid * _SL, _SL)])


_sc_deg = pl.kernel(
    _sc_deg_body,
    out_type=jax.ShapeDtypeStruct((2 * _NP,), jnp.float32),
    mesh=plsc.VectorSubcoreMesh(core_axis_name="c", subcore_axis_name="s"),
    compiler_params=pltpu.CompilerParams(use_tc_tiling_on_sc=False),
    scratch_types=[
        pltpu.VMEM((3, _GED), jnp.int32),
        pltpu.VMEM((_GED,), jnp.float32),
        pltpu.VMEM((_SL,), jnp.float32),
        pltpu.VMEM_SHARED((_NP,), jnp.float32),
        pltpu.SemaphoreType.DMA((3,)),
        pltpu.SemaphoreType.DMA((2,)),
    ],
)


def _sc_edge_body(ef, hn, acc_out, idx, rows, acc_sh, gsem, ssem, isem):
    cid = lax.axis_index("c")
    sid = lax.axis_index("s")
    w = cid * 16 + sid

    def zrow(i, carry):
        rows[0, i, :] = jnp.zeros((16,), jnp.float32)
        return carry

    lax.fori_loop(0, _CK, zrow, 0)
    r0 = sid * _SL

    def zacc(k, carry):
        pltpu.sync_copy(rows.at[0, pl.ds(0, _CK)], acc_sh.at[pl.ds(r0 + k * _CK, _CK)])
        return carry

    lax.fori_loop(0, _ZF, zacc, 0)
    plsc.subcore_barrier()

    base = w * _QGE + jnp.minimum(w, _RGE)
    ng = _QGE + jnp.where(w < _RGE, 1, 0)

    def drain(sem, p):
        pltpu.make_async_copy(acc_out.at[0, pl.ds(0, _GEE)],
                              rows.at[p], sem.at[p]).wait()

    def drain_i(buf):
        for h in range(2):
            pltpu.make_async_copy(ef.at[pl.ds(0, _GEE)], idx.at[buf, h],
                                  isem.at[buf]).wait()

    def load_idx(g, buf):
        pltpu.async_copy(ef.at[pl.ds((base + g) * _GEE, _GEE)],
                         idx.at[buf, 0], isem.at[buf])
        pltpu.async_copy(ef.at[pl.ds(_E + (base + g) * _GEE, _GEE)],
                         idx.at[buf, 1], isem.at[buf])

    def gathers(ib, rb):
        pltpu.async_copy(hn.at[idx.at[ib, 0]], rows.at[rb], gsem.at[rb])

    def scatters(ib, rb):
        pltpu.async_copy(rows.at[rb], acc_sh.at[idx.at[ib, 1]],
                         ssem.at[rb], add=True)

    load_idx(0, 0)
    load_idx(1, 1)
    load_idx(2, 2)
    drain_i(0)
    gathers(0, 0)
    drain_i(1)
    gathers(1, 1)

    def body(g, carry):
        ib = lax.rem(g, 4)
        i2 = lax.rem(g + 2, 4)
        i3 = lax.rem(g + 3, 4)
        rb = lax.rem(g, 3)
        r2 = lax.rem(g + 2, 3)

        @pl.when(g + 2 < ng)
        def _prefetch():
            @pl.when(g >= 1)
            def _():
                drain(ssem, r2)

            @pl.when(g + 3 < ng)
            def _():
                load_idx(g + 3, i3)
            drain_i(i2)
            gathers(i2, r2)

        drain(gsem, rb)
        scatters(ib, rb)
        return carry

    lax.fori_loop(0, ng, body, 0)
    drain(ssem, 0)
    drain(ssem, 1)
    drain(ssem, 2)
    plsc.subcore_barrier()
    pltpu.sync_copy(acc_sh.at[pl.ds(r0, _SL)],
                    acc_out.at[cid, pl.ds(r0, _SL)])


_sc_edge = pl.kernel(
    _sc_edge_body,
    out_type=jax.ShapeDtypeStruct((2, _NP, _H), jnp.float32),
    mesh=plsc.VectorSubcoreMesh(core_axis_name="c", subcore_axis_name="s"),
    compiler_params=pltpu.CompilerParams(use_tc_tiling_on_sc=False),
    scratch_types=[
        pltpu.VMEM((4, 2, _GEE), jnp.int32),
        pltpu.VMEM((3, _GEE, _H), jnp.float32),
        pltpu.VMEM_SHARED((_NP, _H), jnp.float32),
        pltpu.SemaphoreType.DMA((3,)),
        pltpu.SemaphoreType.DMA((3,)),
        pltpu.SemaphoreType.DMA((4,)),
    ],
)

_RB = 2048
_GRID = _NP // _RB

_SQRT_HALF = 0.7071067811865476


def _gelu(t):
    return 0.5 * t * (1.0 + lax.erf(t * _SQRT_HALF))


def _tc_layer0_body(x_ref, w0_ref, b0_ref, g0_ref, bb0_ref, wg_ref, h1_ref, hp_ref):
    h = jnp.dot(x_ref[...], w0_ref[...], preferred_element_type=jnp.float32)
    h = h + b0_ref[...]
    m = jnp.mean(h, axis=1, keepdims=True)
    v = jnp.mean((h - m) ** 2, axis=1, keepdims=True)
    h = (h - m) / jnp.sqrt(v + 1e-5) * g0_ref[...] + bb0_ref[...]
    h1 = _gelu(h)
    h1_ref[...] = h1
    hp_ref[...] = jnp.dot(h1, wg_ref[...], preferred_element_type=jnp.float32)


_tc_layer0 = pl.pallas_call(
    _tc_layer0_body,
    grid=(_GRID,),
    in_specs=[
        pl.BlockSpec((_RB, 128), lambda i: (i, 0)),
        pl.BlockSpec((128, _H), lambda i: (0, 0)),
        pl.BlockSpec((1, _H), lambda i: (0, 0)),
        pl.BlockSpec((1, _H), lambda i: (0, 0)),
        pl.BlockSpec((1, _H), lambda i: (0, 0)),
        pl.BlockSpec((_H, _H), lambda i: (0, 0)),
    ],
    out_specs=[
        pl.BlockSpec((_RB, _H), lambda i: (i, 0)),
        pl.BlockSpec((_RB, _H), lambda i: (i, 0)),
    ],
    out_shape=[
        jax.ShapeDtypeStruct((_N, _H), jnp.float32),
        jax.ShapeDtypeStruct((_N, _H), jnp.float32),
    ],
)


def _dis_col(d):
    t = d[0, :] + d[1, :] + 1.0
    return lax.rsqrt(t).reshape(_RB, 1)


def _tc_hn_body(d_ref, hp_ref, hn_ref):
    hn_ref[...] = hp_ref[...] * _dis_col(d_ref[...])


_tc_hn = pl.pallas_call(
    _tc_hn_body,
    grid=(_GRID,),
    in_specs=[
        pl.BlockSpec((2, _RB), lambda i: (0, i)),
        pl.BlockSpec((_RB, _H), lambda i: (i, 0)),
    ],
    out_specs=pl.BlockSpec((_RB, _H), lambda i: (i, 0)),
    out_shape=jax.ShapeDtypeStruct((_N, _H), jnp.float32),
)


def _tc_final_body(acc_ref, d_ref, hn_ref, h1_ref, bg_ref, g1_ref,
                   b1_ref, w2_ref, b2_ref, out_ref):
    a = acc_ref[0] + acc_ref[1]
    dis = _dis_col(d_ref[...])
    z = (a + hn_ref[...]) * dis + bg_ref[...]
    m = jnp.mean(z, axis=1, keepdims=True)
    v = jnp.mean((z - m) ** 2, axis=1, keepdims=True)
    t = (z - m) / jnp.sqrt(v + 1e-5) * g1_ref[...] + b1_ref[...]
    t = _gelu(t)
    h = t + h1_ref[...]
    out_ref[...] = jnp.dot(h, w2_ref[...], preferred_element_type=jnp.float32) + b2_ref[...]


_tc_final = pl.pallas_call(
    _tc_final_body,
    grid=(_GRID,),
    in_specs=[
        pl.BlockSpec((2, _RB, _H), lambda i: (0, i, 0)),
        pl.BlockSpec((2, _RB), lambda i: (0, i)),
        pl.BlockSpec((_RB, _H), lambda i: (i, 0)),
        pl.BlockSpec((_RB, _H), lambda i: (i, 0)),
        pl.BlockSpec((1, _H), lambda i: (0, 0)),
        pl.BlockSpec((1, _H), lambda i: (0, 0)),
        pl.BlockSpec((1, _H), lambda i: (0, 0)),
        pl.BlockSpec((_H, 128), lambda i: (0, 0)),
        pl.BlockSpec((1, 128), lambda i: (0, 0)),
    ],
    out_specs=pl.BlockSpec((_RB, 128), lambda i: (i, 0)),
    out_shape=jax.ShapeDtypeStruct((_N, 128), jnp.float32),
)


def kernel(x, edge_index, W0, b0, ln0_g, ln0_b, Wg, bg, ln1_g, ln1_b, W2, b2):
    ef = edge_index.reshape(2 * _E)
    deg = _sc_deg(ef)
    h1, hp = _tc_layer0(x, W0, b0.reshape(1, _H), ln0_g.reshape(1, _H),
                        ln0_b.reshape(1, _H), Wg)
    deg2 = deg.reshape(2, _NP)
    hn = _tc_hn(deg2, hp)
    acc = _sc_edge(ef, hn)
    out = _tc_final(acc, deg2, hn, h1, bg.reshape(1, _H),
                    ln1_g.reshape(1, _H), ln1_b.reshape(1, _H),
                    W2, b2.reshape(1, 128))
    return out
